# Initial kernel scaffold; baseline (speedup 1.0000x reference)
import jax
import jax.numpy as jnp
from jax.experimental import pallas as pl


def kernel(x, edge_index, edge_attr, u, batch, W1, b1, W2, b2):
    raise NotImplementedError("write your pallas kernel here")


# speedup vs baseline: 5.4497x; 5.4497x over previous
import functools

import jax
import jax.numpy as jnp
from jax import lax
from jax.experimental import pallas as pl
from jax.experimental.pallas import tpu as pltpu
from jax.experimental.pallas import tpu_sc as plsc

N_NODES = 10000
N_EDGES = 320000
D_FEAT = 128
D_EDGE = 16
D_U = 16
N_GRAPHS = 64
HIDDEN = 256

NTILES = 32
EDGES_PER_TILE = N_EDGES // NTILES
CHUNK = 125
CHUNKS_PER_TILE = EDGES_PER_TILE // CHUNK
STAGE_CHUNKS = 16
N_STAGES = CHUNKS_PER_TILE // STAGE_CHUNKS
NPAD = 10240
ROWS_PER_TILE = NPAD // 16

_sc_mesh = plsc.VectorSubcoreMesh(core_axis_name="c", subcore_axis_name="s")


@functools.partial(
    pl.kernel,
    out_type=(
        jax.ShapeDtypeStruct((2, NPAD, D_EDGE), jnp.float32),
        jax.ShapeDtypeStruct((2, NPAD), jnp.float32),
    ),
    mesh=_sc_mesh,
    compiler_params=pltpu.CompilerParams(use_tc_tiling_on_sc=False),
    scratch_types=[
        pltpu.VMEM((CHUNKS_PER_TILE, CHUNK), jnp.int32),
        pltpu.VMEM((STAGE_CHUNKS, CHUNK, D_EDGE), jnp.float32),
        pltpu.VMEM((128, D_EDGE), jnp.float32),
        pltpu.VMEM((ROWS_PER_TILE,), jnp.float32),
        pltpu.VMEM((128,), jnp.float32),
        pltpu.VMEM_SHARED((NPAD, D_EDGE), jnp.float32),
        pltpu.VMEM_SHARED((NPAD,), jnp.float32),
    ],
)
def _sc_scatter(dest_ref, edges_ref, out_s, out_c,
                idx_v, ebuf, zrows, zflat, ones_v, sums_sp, counts_sp):
    cid = lax.axis_index("c")
    sid = lax.axis_index("s")
    gid = cid * 16 + sid

    zero16 = jnp.zeros((16,), jnp.float32)
    one16 = jnp.ones((16,), jnp.float32)

    def fill_zrows(i, _):
        zrows[i, :] = zero16
        return 0
    lax.fori_loop(0, 128, fill_zrows, 0)

    def fill_zflat(i, _):
        zflat[pl.ds(i * 16, 16)] = zero16
        return 0
    lax.fori_loop(0, ROWS_PER_TILE // 16, fill_zflat, 0)

    def fill_ones(i, _):
        ones_v[pl.ds(i * 16, 16)] = one16
        return 0
    lax.fori_loop(0, 8, fill_ones, 0)

    for r in range(ROWS_PER_TILE // 128):
        pltpu.sync_copy(zrows,
                        sums_sp.at[pl.ds(sid * ROWS_PER_TILE + r * 128, 128)])
    pltpu.sync_copy(zflat, counts_sp.at[pl.ds(sid * ROWS_PER_TILE, ROWS_PER_TILE)])

    pltpu.sync_copy(dest_ref.at[pl.ds(gid * CHUNKS_PER_TILE, CHUNKS_PER_TILE)], idx_v)

    plsc.subcore_barrier()

    for k in range(N_STAGES):
        pltpu.sync_copy(edges_ref.at[pl.ds(gid * CHUNKS_PER_TILE + k * STAGE_CHUNKS,
                                           STAGE_CHUNKS)], ebuf)

        def body(j, _):
            row = idx_v.at[k * STAGE_CHUNKS + j]
            pltpu.sync_copy(ebuf.at[j], sums_sp.at[row], add=True)
            pltpu.sync_copy(ones_v.at[pl.ds(0, CHUNK)],
                            counts_sp.at[row], add=True)
            return 0
        lax.fori_loop(0, STAGE_CHUNKS, body, 0)

    plsc.subcore_barrier()

    pltpu.sync_copy(sums_sp.at[pl.ds(sid * ROWS_PER_TILE, ROWS_PER_TILE)],
                    out_s.at[cid, pl.ds(sid * ROWS_PER_TILE, ROWS_PER_TILE)])
    pltpu.sync_copy(counts_sp.at[pl.ds(sid * ROWS_PER_TILE, ROWS_PER_TILE)],
                    out_c.at[cid, pl.ds(sid * ROWS_PER_TILE, ROWS_PER_TILE)])


_BLK = 1000
_GRID = N_NODES // _BLK


def _mlp_body(x_ref, s0_ref, s1_ref, c0_ref, c1_ref, b_ref, u_ref,
              W1_ref, b1_ref, W2_ref, b2_ref, y_ref, e_ref):
    sums = s0_ref[...] + s1_ref[...]
    counts = jnp.maximum(c0_ref[...] + c1_ref[...], 1.0)
    ebar = sums / counts

    iot = lax.broadcasted_iota(jnp.int32, (_BLK, N_GRAPHS), 1).astype(jnp.float32)
    onehot = (b_ref[...] == iot).astype(jnp.float32)
    ug = jnp.dot(onehot, u_ref[...], preferred_element_type=jnp.float32)

    W1 = W1_ref[...]
    h = (jnp.dot(x_ref[...], W1[:D_FEAT], preferred_element_type=jnp.float32)
         + jnp.dot(ebar, W1[D_FEAT:D_FEAT + D_EDGE], preferred_element_type=jnp.float32)
         + jnp.dot(ug, W1[D_FEAT + D_EDGE:], preferred_element_type=jnp.float32)
         + b1_ref[...])
    h = jnp.maximum(h, 0.0)
    y = jnp.maximum(jnp.dot(h, W2_ref[...], preferred_element_type=jnp.float32)
                    + b2_ref[...], 0.0)
    y_ref[...] = y
    e_ref[...] = ebar


_mlp_call = pl.pallas_call(
    _mlp_body,
    grid=(_GRID,),
    in_specs=[
        pl.BlockSpec((_BLK, D_FEAT), lambda i: (i, 0)),
        pl.BlockSpec((_BLK, D_EDGE), lambda i: (i, 0)),
        pl.BlockSpec((_BLK, D_EDGE), lambda i: (i, 0)),
        pl.BlockSpec((_BLK, 1), lambda i: (i, 0)),
        pl.BlockSpec((_BLK, 1), lambda i: (i, 0)),
        pl.BlockSpec((_BLK, 1), lambda i: (i, 0)),
        pl.BlockSpec((N_GRAPHS, D_U), lambda i: (0, 0)),
        pl.BlockSpec((D_FEAT + D_EDGE + D_U, HIDDEN), lambda i: (0, 0)),
        pl.BlockSpec((1, HIDDEN), lambda i: (0, 0)),
        pl.BlockSpec((HIDDEN, D_FEAT), lambda i: (0, 0)),
        pl.BlockSpec((1, D_FEAT), lambda i: (0, 0)),
    ],
    out_specs=(
        pl.BlockSpec((_BLK, D_FEAT), lambda i: (i, 0)),
        pl.BlockSpec((_BLK, D_EDGE), lambda i: (i, 0)),
    ),
    out_shape=(
        jax.ShapeDtypeStruct((N_NODES, D_FEAT), jnp.float32),
        jax.ShapeDtypeStruct((N_NODES, D_EDGE), jnp.float32),
    ),
)


def kernel(x, edge_index, edge_attr, u, batch, W1, b1, W2, b2):
    dest = edge_index[1].astype(jnp.int32).reshape(N_EDGES // CHUNK, CHUNK)
    edges = edge_attr.reshape(N_EDGES // CHUNK, CHUNK, D_EDGE)
    s_parts, c_parts = _sc_scatter(dest, edges)
    s0 = s_parts[0, :N_NODES]
    s1 = s_parts[1, :N_NODES]
    c0 = c_parts[0, :N_NODES].reshape(N_NODES, 1)
    c1 = c_parts[1, :N_NODES].reshape(N_NODES, 1)
    bf = batch.astype(jnp.float32).reshape(N_NODES, 1)
    y, ebar = _mlp_call(x, s0, s1, c0, c1, bf, u,
                        W1, b1.reshape(1, HIDDEN), W2, b2.reshape(1, D_FEAT))
    return (y, ebar)


# speedup vs baseline: 6.1545x; 1.1293x over previous
import functools

import jax
import jax.numpy as jnp
from jax import lax
from jax.experimental import pallas as pl
from jax.experimental.pallas import tpu as pltpu
from jax.experimental.pallas import tpu_sc as plsc

N_NODES = 10000
N_EDGES = 320000
D_FEAT = 128
D_EDGE = 16
D_U = 16
N_GRAPHS = 64
HIDDEN = 256

NTILES = 32
EDGES_PER_TILE = N_EDGES // NTILES
CHUNK = 125
CHUNKS_PER_TILE = EDGES_PER_TILE // CHUNK
STAGE_CHUNKS = 16
N_STAGES = CHUNKS_PER_TILE // STAGE_CHUNKS
NPAD = 10240
ROWS_PER_TILE = NPAD // 16

_sc_mesh = plsc.VectorSubcoreMesh(core_axis_name="c", subcore_axis_name="s")


@functools.partial(
    pl.kernel,
    out_type=(
        jax.ShapeDtypeStruct((2, NPAD, D_EDGE), jnp.float32),
        jax.ShapeDtypeStruct((2, NPAD), jnp.float32),
    ),
    mesh=_sc_mesh,
    compiler_params=pltpu.CompilerParams(use_tc_tiling_on_sc=False),
    scratch_types=[
        pltpu.VMEM((CHUNKS_PER_TILE, CHUNK), jnp.int32),
        pltpu.VMEM((STAGE_CHUNKS * CHUNK, D_EDGE), jnp.float32),
        pltpu.VMEM((128, D_EDGE), jnp.float32),
        pltpu.VMEM((ROWS_PER_TILE,), jnp.float32),
        pltpu.VMEM((128,), jnp.float32),
        pltpu.VMEM_SHARED((NPAD, D_EDGE), jnp.float32),
        pltpu.VMEM_SHARED((NPAD,), jnp.float32),
    ],
)
def _sc_scatter(dest_ref, edges_ref, out_s, out_c,
                idx_v, ebuf, zrows, zflat, ones_v, sums_sp, counts_sp):
    cid = lax.axis_index("c")
    sid = lax.axis_index("s")
    gid = cid * 16 + sid

    zero16 = jnp.zeros((16,), jnp.float32)
    one16 = jnp.ones((16,), jnp.float32)

    def fill_zrows(i, _):
        zrows[i, :] = zero16
        return 0
    lax.fori_loop(0, 128, fill_zrows, 0)

    def fill_zflat(i, _):
        zflat[pl.ds(i * 16, 16)] = zero16
        return 0
    lax.fori_loop(0, ROWS_PER_TILE // 16, fill_zflat, 0)

    def fill_ones(i, _):
        ones_v[pl.ds(i * 16, 16)] = one16
        return 0
    lax.fori_loop(0, 8, fill_ones, 0)

    for r in range(ROWS_PER_TILE // 128):
        pltpu.sync_copy(zrows,
                        sums_sp.at[pl.ds(sid * ROWS_PER_TILE + r * 128, 128)])
    pltpu.sync_copy(zflat, counts_sp.at[pl.ds(sid * ROWS_PER_TILE, ROWS_PER_TILE)])

    pltpu.sync_copy(dest_ref.at[pl.ds(gid * CHUNKS_PER_TILE, CHUNKS_PER_TILE)], idx_v)

    plsc.subcore_barrier()

    for k in range(N_STAGES):
        pltpu.sync_copy(
            edges_ref.at[pl.ds(gid * EDGES_PER_TILE + k * STAGE_CHUNKS * CHUNK,
                               STAGE_CHUNKS * CHUNK)], ebuf)

        def body(j, _):
            row = idx_v.at[k * STAGE_CHUNKS + j]
            pltpu.sync_copy(ebuf.at[pl.ds(j * CHUNK, CHUNK)], sums_sp.at[row], add=True)
            pltpu.sync_copy(ones_v.at[pl.ds(0, CHUNK)],
                            counts_sp.at[row], add=True)
            return 0
        lax.fori_loop(0, STAGE_CHUNKS, body, 0)

    plsc.subcore_barrier()

    pltpu.sync_copy(sums_sp.at[pl.ds(sid * ROWS_PER_TILE, ROWS_PER_TILE)],
                    out_s.at[cid, pl.ds(sid * ROWS_PER_TILE, ROWS_PER_TILE)])
    pltpu.sync_copy(counts_sp.at[pl.ds(sid * ROWS_PER_TILE, ROWS_PER_TILE)],
                    out_c.at[cid, pl.ds(sid * ROWS_PER_TILE, ROWS_PER_TILE)])


_BLK = 1000
_GRID = N_NODES // _BLK


def _mlp_body(x_ref, s0_ref, s1_ref, c0_ref, c1_ref, b_ref, u_ref,
              W1_ref, b1_ref, W2_ref, b2_ref, y_ref, e_ref):
    sums = s0_ref[...] + s1_ref[...]
    counts = jnp.maximum(c0_ref[...] + c1_ref[...], 1.0)
    ebar = sums / counts

    iot = lax.broadcasted_iota(jnp.int32, (_BLK, N_GRAPHS), 1).astype(jnp.float32)
    onehot = (b_ref[...] == iot).astype(jnp.float32)
    ug = jnp.dot(onehot, u_ref[...], preferred_element_type=jnp.float32)

    W1 = W1_ref[...]
    h = (jnp.dot(x_ref[...], W1[:D_FEAT], preferred_element_type=jnp.float32)
         + jnp.dot(ebar, W1[D_FEAT:D_FEAT + D_EDGE], preferred_element_type=jnp.float32)
         + jnp.dot(ug, W1[D_FEAT + D_EDGE:], preferred_element_type=jnp.float32)
         + b1_ref[...])
    h = jnp.maximum(h, 0.0)
    y = jnp.maximum(jnp.dot(h, W2_ref[...], preferred_element_type=jnp.float32)
                    + b2_ref[...], 0.0)
    y_ref[...] = y
    e_ref[...] = ebar


_mlp_call = pl.pallas_call(
    _mlp_body,
    grid=(_GRID,),
    in_specs=[
        pl.BlockSpec((_BLK, D_FEAT), lambda i: (i, 0)),
        pl.BlockSpec((_BLK, D_EDGE), lambda i: (i, 0)),
        pl.BlockSpec((_BLK, D_EDGE), lambda i: (i, 0)),
        pl.BlockSpec((_BLK, 1), lambda i: (i, 0)),
        pl.BlockSpec((_BLK, 1), lambda i: (i, 0)),
        pl.BlockSpec((_BLK, 1), lambda i: (i, 0)),
        pl.BlockSpec((N_GRAPHS, D_U), lambda i: (0, 0)),
        pl.BlockSpec((D_FEAT + D_EDGE + D_U, HIDDEN), lambda i: (0, 0)),
        pl.BlockSpec((1, HIDDEN), lambda i: (0, 0)),
        pl.BlockSpec((HIDDEN, D_FEAT), lambda i: (0, 0)),
        pl.BlockSpec((1, D_FEAT), lambda i: (0, 0)),
    ],
    out_specs=(
        pl.BlockSpec((_BLK, D_FEAT), lambda i: (i, 0)),
        pl.BlockSpec((_BLK, D_EDGE), lambda i: (i, 0)),
    ),
    out_shape=(
        jax.ShapeDtypeStruct((N_NODES, D_FEAT), jnp.float32),
        jax.ShapeDtypeStruct((N_NODES, D_EDGE), jnp.float32),
    ),
)


def kernel(x, edge_index, edge_attr, u, batch, W1, b1, W2, b2):
    dest = edge_index[1].astype(jnp.int32).reshape(N_EDGES // CHUNK, CHUNK)
    s_parts, c_parts = _sc_scatter(dest, edge_attr)
    s0 = s_parts[0, :N_NODES]
    s1 = s_parts[1, :N_NODES]
    c0 = c_parts[0, :N_NODES].reshape(N_NODES, 1)
    c1 = c_parts[1, :N_NODES].reshape(N_NODES, 1)
    bf = batch.astype(jnp.float32).reshape(N_NODES, 1)
    y, ebar = _mlp_call(x, s0, s1, c0, c1, bf, u,
                        W1, b1.reshape(1, HIDDEN), W2, b2.reshape(1, D_FEAT))
    return (y, ebar)
